# baseline (speedup 1.0000x reference)
import functools

import jax
import jax.numpy as jnp
from jax import lax
from jax.experimental import pallas as pl
from jax.experimental.pallas import tpu as pltpu
from jax.experimental.pallas import tpu_sc as plsc

N = 10000
E = 320000
D = 128
H1 = 8
O1 = 8
C = 40

NP = 10112
NCORES = 2
NSUB = 16
NW = NCORES * NSUB
CH = 96
E_TOT = E + N
NB = 4
CPS = -(-(-(-E_TOT // (NW * CH))) // 12) * 12
E_PAD = CPS * NW * CH
ROWS_PER_SUB = NP // NSUB

WA1 = 80
WA2 = 48


def _sc_edge_kernel(wa, body, nb):
    mesh = plsc.VectorSubcoreMesh(core_axis_name="c", subcore_axis_name="s")

    @functools.partial(
        pl.kernel,
        out_type=jax.ShapeDtypeStruct((NCORES, NP, wa), jnp.float32),
        mesh=mesh,
        scratch_types=[
            pltpu.VMEM((nb, 2, CH), jnp.int32),
            pltpu.VMEM((nb, CH, wa), jnp.float32),
            pltpu.VMEM((nb, CH, 16), jnp.float32),
            pltpu.VMEM((nb, CH, wa), jnp.float32),
            pltpu.VMEM_SHARED((NP, wa), jnp.float32),
            pltpu.SemaphoreType.DMA((nb,)),
            pltpu.SemaphoreType.DMA((nb,)),
            pltpu.SemaphoreType.DMA((nb,)),
            pltpu.SemaphoreType.DMA((nb,)),
        ],
        compiler_params=pltpu.CompilerParams(use_tc_tiling_on_sc=False),
    )
    def kern(edges, tab_a, tab_b, zeros, out, idx_v, a_v, b_v, c_v, accum,
             sem_i, sem_a, sem_b, sem_s):
        ci = lax.axis_index("c")
        si = lax.axis_index("s")
        wid = ci * NSUB + si

        r0 = si * ROWS_PER_SUB
        pltpu.sync_copy(zeros.at[pl.ds(r0, ROWS_PER_SUB)],
                        accum.at[pl.ds(r0, ROWS_PER_SUB)])
        plsc.subcore_barrier()

        cbase = wid * CPS

        def issue_i(g, b):
            pltpu.async_copy(edges.at[cbase + g], idx_v.at[b], sem_i.at[b])

        def wait_i(b):
            pltpu.make_async_copy(edges.at[cbase], idx_v.at[b],
                                  sem_i.at[b]).wait()

        def issue_g(b):
            pltpu.async_copy(tab_a.at[idx_v.at[b, 0]], a_v.at[b], sem_a.at[b])
            pltpu.async_copy(tab_b.at[idx_v.at[b, 1]], b_v.at[b], sem_b.at[b])

        def wait_g(b):
            pltpu.make_async_copy(tab_a.at[idx_v.at[b, 0]], a_v.at[b],
                                  sem_a.at[b]).wait()
            pltpu.make_async_copy(tab_b.at[idx_v.at[b, 1]], b_v.at[b],
                                  sem_b.at[b]).wait()

        def issue_s(b):
            pltpu.async_copy(c_v.at[b], accum.at[idx_v.at[b, 1]],
                             sem_s.at[b], add=True)

        def wait_s(b):
            pltpu.make_async_copy(c_v.at[b], accum.at[idx_v.at[b, 1]],
                                  sem_s.at[b]).wait()

        issue_i(0, 0)
        issue_i(1, 1)
        wait_i(0)
        issue_g(0)

        def super_step(s, carry):
            g0 = s * nb
            for k in range(nb):
                g = g0 + k

                @pl.when(g >= nb - 2)
                def _():
                    wait_s((k + 2) % nb)

                @pl.when(g + 1 < CPS)
                def _():
                    wait_i((k + 1) % nb)
                    issue_g((k + 1) % nb)

                @pl.when(g + 2 < CPS)
                def _():
                    issue_i(g + 2, (k + 2) % nb)

                wait_g(k)

                @plsc.parallel_loop(0, CH, 1, unroll=8)
                def _(e, _k=k):
                    body(e, a_v.at[_k], b_v.at[_k], c_v.at[_k])

                issue_s(k)
            return carry

        lax.fori_loop(0, CPS // nb, super_step, 0)
        for j in range(nb - 2):
            wait_s((CPS - 1 - j) % nb)

        plsc.subcore_barrier()
        pltpu.sync_copy(accum.at[pl.ds(r0, ROWS_PER_SUB)],
                        out.at[ci, pl.ds(r0, ROWS_PER_SUB)])

    return kern


def _edge_body1(e, a_v, b_v, c_v):
    a4 = a_v[e, pl.ds(64, 16)]
    b = b_v[e, pl.ds(0, 16)]
    u = a4 + b
    t = jnp.exp(jnp.where(u >= 0.0, u, u * 0.2))
    c_v[e, pl.ds(0, 16)] = a_v[e, pl.ds(0, 16)] * t
    c_v[e, pl.ds(16, 16)] = a_v[e, pl.ds(16, 16)] * t
    c_v[e, pl.ds(32, 16)] = a_v[e, pl.ds(32, 16)] * t
    c_v[e, pl.ds(48, 16)] = a_v[e, pl.ds(48, 16)] * t
    io = lax.iota(jnp.int32, 16)
    c_v[e, pl.ds(64, 16)] = jnp.where(io < 8, t, 0.0)


def _edge_body2(e, a_v, b_v, c_v):
    a2 = a_v[e, pl.ds(32, 16)]
    b = b_v[e, pl.ds(0, 16)]
    u = a2 + b
    t8 = jnp.exp(jnp.where(u >= 0.0, u, u * 0.2))
    tr = lax.rev(t8, (0,))
    io = lax.iota(jnp.int32, 16)
    t = jnp.where(io < 8, tr, t8)
    c_v[e, pl.ds(0, 16)] = a_v[e, pl.ds(0, 16)] * t
    c_v[e, pl.ds(16, 16)] = a_v[e, pl.ds(16, 16)] * t
    c_v[e, pl.ds(32, 16)] = jnp.where(io < 8, a2 * t,
                                      jnp.where(io == 8, t, 0.0))


_sc_layer1 = _sc_edge_kernel(WA1, _edge_body1, 4)
_sc_layer2 = _sc_edge_kernel(WA2, _edge_body2, 6)


def _tc_a(x_ref, w_ref, ast_ref, adt_ref, taba_ref, tabb_ref):
    xp = jnp.dot(x_ref[...], w_ref[...], preferred_element_type=jnp.float32)
    ys = xp * ast_ref[...]
    yd = xp * adt_ref[...]
    as8 = sum(ys[:, 8 * c:8 * c + 8] for c in range(8))
    ad8 = sum(yd[:, 8 * c:8 * c + 8] for c in range(8))
    taba_ref[...] = jnp.concatenate([xp, as8, as8], axis=1)
    tabb_ref[...] = jnp.concatenate([ad8, ad8], axis=1)


def _tc_b(acc_ref, b1t_ref, w2p_ref, a2s_ref, a2d_ref, taba_ref, tabb_ref):
    p = acc_ref[0] + acc_ref[1]
    den8 = p[:, 64:72] + 1e-16
    den64 = jnp.concatenate([den8] * 8, axis=1)
    h = p[:, :64] / den64 + b1t_ref[...]
    h = jnp.where(h > 0.0, h, jnp.exp(jnp.minimum(h, 0.0)) - 1.0)
    xp2 = jnp.dot(h, w2p_ref[...], preferred_element_type=jnp.float32)
    s = jnp.sum(xp2 * a2s_ref[...], axis=1, keepdims=True)
    d = jnp.sum(xp2 * a2d_ref[...], axis=1, keepdims=True)
    bn = xp2.shape[0]
    taba_ref[...] = jnp.concatenate(
        [xp2, jnp.broadcast_to(s, (bn, 8))], axis=1)
    tabb_ref[...] = jnp.broadcast_to(d, (bn, 16))


def _tc_c(acc_ref, b2_ref, out_ref):
    p = acc_ref[0] + acc_ref[1]
    den = p[:, 40:41] + 1e-16
    o = p[:, :40] / den + b2_ref[...]
    o = jnp.where(o > 0.0, o, jnp.exp(jnp.minimum(o, 0.0)) - 1.0)
    m = jnp.max(o, axis=1, keepdims=True)
    z = o - m
    out_ref[...] = z - jnp.log(jnp.sum(jnp.exp(z), axis=1, keepdims=True))


_GB = 8
_BN = NP // _GB


def _row_spec(w):
    return pl.BlockSpec((_BN, w), lambda i: (i, 0))


def _full_spec(shape):
    return pl.BlockSpec(shape, lambda i: (0,) * len(shape))


_tc_a_call = pl.pallas_call(
    _tc_a,
    grid=(_GB,),
    in_specs=[_row_spec(D), _full_spec((D, 64)), _full_spec((1, 64)),
              _full_spec((1, 64))],
    out_specs=[_row_spec(WA1), _row_spec(16)],
    out_shape=[jax.ShapeDtypeStruct((NP, WA1), jnp.float32),
               jax.ShapeDtypeStruct((NP, 16), jnp.float32)])
_tc_b_call = pl.pallas_call(
    _tc_b,
    grid=(_GB,),
    in_specs=[pl.BlockSpec((NCORES, _BN, WA1), lambda i: (0, i, 0)),
              _full_spec((1, 64)), _full_spec((64, C)), _full_spec((1, C)),
              _full_spec((1, C))],
    out_specs=[_row_spec(WA2), _row_spec(16)],
    out_shape=[jax.ShapeDtypeStruct((NP, WA2), jnp.float32),
               jax.ShapeDtypeStruct((NP, 16), jnp.float32)])
_tc_c_call = pl.pallas_call(
    _tc_c,
    grid=(_GB,),
    in_specs=[pl.BlockSpec((NCORES, _BN, WA2), lambda i: (0, i, 0)),
              _full_spec((1, C))],
    out_specs=_row_spec(C),
    out_shape=jax.ShapeDtypeStruct((NP, C), jnp.float32))


def kernel(x, edge_index, W1, att_src1, att_dst1, b1, W2, att_src2, att_dst2,
           b2):
    w1p = W1.reshape(D, H1, O1).transpose(0, 2, 1).reshape(D, H1 * O1)
    ast = att_src1.reshape(H1, O1).T.reshape(1, H1 * O1)
    adt = att_dst1.reshape(H1, O1).T.reshape(1, H1 * O1)
    b1t = b1.reshape(H1, O1).T.reshape(1, H1 * O1)
    w2p = W2.reshape(H1, O1, C).transpose(1, 0, 2).reshape(H1 * O1, C)
    a2s = att_src2.reshape(1, C)
    a2d = att_dst2.reshape(1, C)
    b2r = b2.reshape(1, C)

    x_pad = jnp.zeros((NP, D), jnp.float32).at[:N].set(x)
    loop = jnp.arange(N, dtype=jnp.int32)
    padv = jnp.full((E_PAD - E_TOT,), N, dtype=jnp.int32)
    src = jnp.concatenate([edge_index[0], loop, padv]).reshape(-1, 1, CH)
    dst = jnp.concatenate([edge_index[1], loop, padv]).reshape(-1, 1, CH)
    edges = jnp.concatenate([src, dst], axis=1)
    z1 = jnp.zeros((NP, WA1), jnp.float32)
    z2 = jnp.zeros((NP, WA2), jnp.float32)

    tab_a1, tab_b1 = _tc_a_call(x_pad, w1p, ast, adt)
    acc1 = _sc_layer1(edges, tab_a1, tab_b1, z1)
    tab_a2, tab_b2 = _tc_b_call(acc1, b1t, w2p, a2s, a2d)
    acc2 = _sc_layer2(edges, tab_a2, tab_b2, z2)
    out = _tc_c_call(acc2, b2r)
    return out[:N]


# speedup vs baseline: 1.0325x; 1.0325x over previous
import functools

import jax
import jax.numpy as jnp
from jax import lax
from jax.experimental import pallas as pl
from jax.experimental.pallas import tpu as pltpu
from jax.experimental.pallas import tpu_sc as plsc

N = 10000
E = 320000
D = 128
H1 = 8
O1 = 8
C = 40

NP = 10112
NCORES = 2
NSUB = 16
NW = NCORES * NSUB
CH = 96
E_TOT = E + N
NB = 4
CPS = -(-(-(-E_TOT // (NW * CH))) // 12) * 12
E_PAD = CPS * NW * CH
ROWS_PER_SUB = NP // NSUB

WA1 = 80
WA2 = 48


def _sc_edge_kernel(wa, body, nb):
    mesh = plsc.VectorSubcoreMesh(core_axis_name="c", subcore_axis_name="s")

    @functools.partial(
        pl.kernel,
        out_type=jax.ShapeDtypeStruct((NCORES, NP, wa), jnp.float32),
        mesh=mesh,
        scratch_types=[
            pltpu.VMEM((12, 2, CH), jnp.int32),
            pltpu.VMEM((nb, CH, wa), jnp.float32),
            pltpu.VMEM((nb, CH, 16), jnp.float32),
            pltpu.VMEM((nb, CH, wa), jnp.float32),
            pltpu.VMEM_SHARED((NP, wa), jnp.float32),
            pltpu.SemaphoreType.DMA((12,)),
            pltpu.SemaphoreType.DMA((nb,)),
            pltpu.SemaphoreType.DMA((nb,)),
            pltpu.SemaphoreType.DMA((nb,)),
        ],
        compiler_params=pltpu.CompilerParams(use_tc_tiling_on_sc=False),
    )
    def kern(edges, tab_a, tab_b, zeros, out, idx_v, a_v, b_v, c_v, accum,
             sem_i, sem_a, sem_b, sem_s):
        ci = lax.axis_index("c")
        si = lax.axis_index("s")
        wid = ci * NSUB + si

        r0 = si * ROWS_PER_SUB
        pltpu.sync_copy(zeros.at[pl.ds(r0, ROWS_PER_SUB)],
                        accum.at[pl.ds(r0, ROWS_PER_SUB)])
        plsc.subcore_barrier()

        cbase = wid * CPS

        def issue_i(g, b):
            pltpu.async_copy(edges.at[cbase + g], idx_v.at[b], sem_i.at[b])

        def wait_i(b):
            pltpu.make_async_copy(edges.at[cbase], idx_v.at[b],
                                  sem_i.at[b]).wait()

        def issue_g(bi, b):
            pltpu.async_copy(tab_a.at[idx_v.at[bi, 0]], a_v.at[b],
                             sem_a.at[b])
            pltpu.async_copy(tab_b.at[idx_v.at[bi, 1]], b_v.at[b],
                             sem_b.at[b])

        def wait_g(b):
            pltpu.make_async_copy(tab_a.at[idx_v.at[0, 0]], a_v.at[b],
                                  sem_a.at[b]).wait()
            pltpu.make_async_copy(tab_b.at[idx_v.at[0, 1]], b_v.at[b],
                                  sem_b.at[b]).wait()

        def issue_s(bi, b):
            pltpu.async_copy(c_v.at[b], accum.at[idx_v.at[bi, 1]],
                             sem_s.at[b], add=True)

        def wait_s(b):
            pltpu.make_async_copy(c_v.at[b], accum.at[idx_v.at[0, 1]],
                                  sem_s.at[b]).wait()

        issue_i(0, 0)
        issue_i(1, 1)
        issue_i(2, 2)
        wait_i(0)
        issue_g(0, 0)
        wait_i(1)
        issue_g(1, 1 % nb)

        def super_step(s, carry):
            g0 = s * 12
            for k in range(12):
                g = g0 + k

                @pl.when(g >= nb - 2)
                def _():
                    wait_s((k + 2) % nb)

                @pl.when(g + 2 < CPS)
                def _():
                    wait_i((k + 2) % 12)
                    issue_g((k + 2) % 12, (k + 2) % nb)

                @pl.when(g + 3 < CPS)
                def _():
                    issue_i(g + 3, (k + 3) % 12)

                wait_g(k % nb)

                @plsc.parallel_loop(0, CH, 1, unroll=8)
                def _(e, _k=k % nb):
                    body(e, a_v.at[_k], b_v.at[_k], c_v.at[_k])

                issue_s(k % 12, k % nb)
            return carry

        lax.fori_loop(0, CPS // 12, super_step, 0)
        for j in range(nb - 2):
            wait_s((CPS - 1 - j) % nb)

        plsc.subcore_barrier()
        pltpu.sync_copy(accum.at[pl.ds(r0, ROWS_PER_SUB)],
                        out.at[ci, pl.ds(r0, ROWS_PER_SUB)])

    return kern


def _edge_body1(e, a_v, b_v, c_v):
    a4 = a_v[e, pl.ds(64, 16)]
    b = b_v[e, pl.ds(0, 16)]
    u = a4 + b
    t = jnp.exp(jnp.where(u >= 0.0, u, u * 0.2))
    c_v[e, pl.ds(0, 16)] = a_v[e, pl.ds(0, 16)] * t
    c_v[e, pl.ds(16, 16)] = a_v[e, pl.ds(16, 16)] * t
    c_v[e, pl.ds(32, 16)] = a_v[e, pl.ds(32, 16)] * t
    c_v[e, pl.ds(48, 16)] = a_v[e, pl.ds(48, 16)] * t
    io = lax.iota(jnp.int32, 16)
    c_v[e, pl.ds(64, 16)] = jnp.where(io < 8, t, 0.0)


def _edge_body2(e, a_v, b_v, c_v):
    a2 = a_v[e, pl.ds(32, 16)]
    b = b_v[e, pl.ds(0, 16)]
    u = a2 + b
    t8 = jnp.exp(jnp.where(u >= 0.0, u, u * 0.2))
    tr = lax.rev(t8, (0,))
    io = lax.iota(jnp.int32, 16)
    t = jnp.where(io < 8, tr, t8)
    c_v[e, pl.ds(0, 16)] = a_v[e, pl.ds(0, 16)] * t
    c_v[e, pl.ds(16, 16)] = a_v[e, pl.ds(16, 16)] * t
    c_v[e, pl.ds(32, 16)] = jnp.where(io < 8, a2 * t,
                                      jnp.where(io == 8, t, 0.0))


_sc_layer1 = _sc_edge_kernel(WA1, _edge_body1, 4)
_sc_layer2 = _sc_edge_kernel(WA2, _edge_body2, 6)


def _tc_a(x_ref, w_ref, ast_ref, adt_ref, taba_ref, tabb_ref):
    xp = jnp.dot(x_ref[...], w_ref[...], preferred_element_type=jnp.float32)
    ys = xp * ast_ref[...]
    yd = xp * adt_ref[...]
    as8 = sum(ys[:, 8 * c:8 * c + 8] for c in range(8))
    ad8 = sum(yd[:, 8 * c:8 * c + 8] for c in range(8))
    taba_ref[...] = jnp.concatenate([xp, as8, as8], axis=1)
    tabb_ref[...] = jnp.concatenate([ad8, ad8], axis=1)


def _tc_b(acc_ref, b1t_ref, w2p_ref, a2s_ref, a2d_ref, taba_ref, tabb_ref):
    p = acc_ref[0] + acc_ref[1]
    den8 = p[:, 64:72] + 1e-16
    den64 = jnp.concatenate([den8] * 8, axis=1)
    h = p[:, :64] / den64 + b1t_ref[...]
    h = jnp.where(h > 0.0, h, jnp.exp(jnp.minimum(h, 0.0)) - 1.0)
    xp2 = jnp.dot(h, w2p_ref[...], preferred_element_type=jnp.float32)
    s = jnp.sum(xp2 * a2s_ref[...], axis=1, keepdims=True)
    d = jnp.sum(xp2 * a2d_ref[...], axis=1, keepdims=True)
    bn = xp2.shape[0]
    taba_ref[...] = jnp.concatenate(
        [xp2, jnp.broadcast_to(s, (bn, 8))], axis=1)
    tabb_ref[...] = jnp.broadcast_to(d, (bn, 16))


def _tc_c(acc_ref, b2_ref, out_ref):
    p = acc_ref[0] + acc_ref[1]
    den = p[:, 40:41] + 1e-16
    o = p[:, :40] / den + b2_ref[...]
    o = jnp.where(o > 0.0, o, jnp.exp(jnp.minimum(o, 0.0)) - 1.0)
    m = jnp.max(o, axis=1, keepdims=True)
    z = o - m
    out_ref[...] = z - jnp.log(jnp.sum(jnp.exp(z), axis=1, keepdims=True))


_GB = 8
_BN = NP // _GB


def _row_spec(w):
    return pl.BlockSpec((_BN, w), lambda i: (i, 0))


def _full_spec(shape):
    return pl.BlockSpec(shape, lambda i: (0,) * len(shape))


_tc_a_call = pl.pallas_call(
    _tc_a,
    grid=(_GB,),
    in_specs=[_row_spec(D), _full_spec((D, 64)), _full_spec((1, 64)),
              _full_spec((1, 64))],
    out_specs=[_row_spec(WA1), _row_spec(16)],
    out_shape=[jax.ShapeDtypeStruct((NP, WA1), jnp.float32),
               jax.ShapeDtypeStruct((NP, 16), jnp.float32)])
_tc_b_call = pl.pallas_call(
    _tc_b,
    grid=(_GB,),
    in_specs=[pl.BlockSpec((NCORES, _BN, WA1), lambda i: (0, i, 0)),
              _full_spec((1, 64)), _full_spec((64, C)), _full_spec((1, C)),
              _full_spec((1, C))],
    out_specs=[_row_spec(WA2), _row_spec(16)],
    out_shape=[jax.ShapeDtypeStruct((NP, WA2), jnp.float32),
               jax.ShapeDtypeStruct((NP, 16), jnp.float32)])
_tc_c_call = pl.pallas_call(
    _tc_c,
    grid=(_GB,),
    in_specs=[pl.BlockSpec((NCORES, _BN, WA2), lambda i: (0, i, 0)),
              _full_spec((1, C))],
    out_specs=_row_spec(C),
    out_shape=jax.ShapeDtypeStruct((NP, C), jnp.float32))


def kernel(x, edge_index, W1, att_src1, att_dst1, b1, W2, att_src2, att_dst2,
           b2):
    w1p = W1.reshape(D, H1, O1).transpose(0, 2, 1).reshape(D, H1 * O1)
    ast = att_src1.reshape(H1, O1).T.reshape(1, H1 * O1)
    adt = att_dst1.reshape(H1, O1).T.reshape(1, H1 * O1)
    b1t = b1.reshape(H1, O1).T.reshape(1, H1 * O1)
    w2p = W2.reshape(H1, O1, C).transpose(1, 0, 2).reshape(H1 * O1, C)
    a2s = att_src2.reshape(1, C)
    a2d = att_dst2.reshape(1, C)
    b2r = b2.reshape(1, C)

    x_pad = jnp.zeros((NP, D), jnp.float32).at[:N].set(x)
    loop = jnp.arange(N, dtype=jnp.int32)
    padv = jnp.full((E_PAD - E_TOT,), N, dtype=jnp.int32)
    src = jnp.concatenate([edge_index[0], loop, padv]).reshape(-1, 1, CH)
    dst = jnp.concatenate([edge_index[1], loop, padv]).reshape(-1, 1, CH)
    edges = jnp.concatenate([src, dst], axis=1)
    z1 = jnp.zeros((NP, WA1), jnp.float32)
    z2 = jnp.zeros((NP, WA2), jnp.float32)

    tab_a1, tab_b1 = _tc_a_call(x_pad, w1p, ast, adt)
    acc1 = _sc_layer1(edges, tab_a1, tab_b1, z1)
    tab_a2, tab_b2 = _tc_b_call(acc1, b1t, w2p, a2s, a2d)
    acc2 = _sc_layer2(edges, tab_a2, tab_b2, z2)
    out = _tc_c_call(acc2, b2r)
    return out[:N]
